# baseline (speedup 1.0000x reference)
import functools

import jax
import jax.numpy as jnp
from jax import lax
from jax.experimental import pallas as pl
from jax.experimental.pallas import tpu as pltpu
from jax.experimental.pallas import tpu_sc as plsc

_info = plsc.get_sparse_core_info()
_NC, _NS = _info.num_cores, _info.num_subcores
_NW = _NC * _NS


def _make_gather(n_total):
    assert n_total % (8 * _NW) == 0
    per = n_total // _NW
    mesh = plsc.VectorSubcoreMesh(core_axis_name="c", subcore_axis_name="s")

    @functools.partial(
        pl.kernel,
        mesh=mesh,
        out_type=jax.ShapeDtypeStruct((n_total,), jnp.float32),
        scratch_types=[
            pltpu.VMEM((per,), jnp.int32),
            pltpu.VMEM((per,), jnp.float32),
            pltpu.SemaphoreType.DMA,
        ],
    )
    def gather_kernel(table_hbm, idx_hbm, out_hbm, idx_v, vals_v, sem):
        wid = lax.axis_index("s") * _NC + lax.axis_index("c")
        base = wid * per
        pltpu.sync_copy(idx_hbm.at[pl.ds(base, per)], idx_v)
        pltpu.async_copy(table_hbm.at[idx_v], vals_v, sem).wait()
        pltpu.sync_copy(vals_v, out_hbm.at[pl.ds(base, per)])

    return gather_kernel


@jax.jit
def kernel(values, data_bias):
    b, f = values.shape
    n = b * f
    nv = data_bias.shape[0]
    idx = values.T.reshape(n).astype(jnp.int32)
    t2 = jax.lax.optimization_barrier(data_bias.T.reshape(nv // 64, 64))
    table = t2.reshape(nv)
    out = _make_gather(n)(table, idx)
    return out.reshape(f, b).T


# speedup vs baseline: 1.1417x; 1.1417x over previous
import functools

import jax
import jax.numpy as jnp
from jax import lax
from jax.experimental import pallas as pl
from jax.experimental.pallas import tpu as pltpu
from jax.experimental.pallas import tpu_sc as plsc

_info = plsc.get_sparse_core_info()
_NC, _NS = _info.num_cores, _info.num_subcores
_NW = _NC * _NS


def _make_gather(n_total):
    assert n_total % (16 * _NW) == 0
    per = n_total // _NW
    half = per // 2
    mesh = plsc.VectorSubcoreMesh(core_axis_name="c", subcore_axis_name="s")

    @functools.partial(
        pl.kernel,
        mesh=mesh,
        out_type=jax.ShapeDtypeStruct((n_total,), jnp.float32),
        scratch_types=[
            pltpu.VMEM((half,), jnp.int32),
            pltpu.VMEM((half,), jnp.int32),
            pltpu.VMEM((half,), jnp.float32),
            pltpu.VMEM((half,), jnp.float32),
            pltpu.SemaphoreType.DMA,
            pltpu.SemaphoreType.DMA,
        ],
    )
    def gather_kernel(
        table_hbm, idx_hbm, out_hbm, idx_a, idx_b, vals_a, vals_b, sem_a,
        sem_b
    ):
        wid = lax.axis_index("s") * _NC + lax.axis_index("c")
        base_a = wid * per
        base_b = base_a + half
        pltpu.sync_copy(idx_hbm.at[pl.ds(base_a, half)], idx_a)
        ga = pltpu.async_copy(table_hbm.at[idx_a], vals_a, sem_a)
        pltpu.sync_copy(idx_hbm.at[pl.ds(base_b, half)], idx_b)
        ga.wait()
        gb = pltpu.async_copy(table_hbm.at[idx_b], vals_b, sem_b)
        pltpu.sync_copy(vals_a, out_hbm.at[pl.ds(base_a, half)])
        gb.wait()
        pltpu.sync_copy(vals_b, out_hbm.at[pl.ds(base_b, half)])

    return gather_kernel


@jax.jit
def kernel(values, data_bias):
    b, f = values.shape
    n = b * f
    idx = values.T.reshape(n).astype(jnp.int32)
    table = data_bias.T.reshape(data_bias.shape[0])
    out = _make_gather(n)(table, idx)
    return out.reshape(f, b).T
